# baseline (speedup 1.0000x reference)
import functools

import jax
import jax.numpy as jnp
from jax import lax
from jax.experimental import pallas as pl
from jax.experimental.pallas import tpu as pltpu
from jax.experimental.pallas import tpu_sc as plsc

B = 16384
F_DENSE = 13
F_SPARSE = 26
VOCAB = 100000
EMB = 16
ONEHOT = 2600
FH = F_SPARSE // 2
TOTAL_H = B * FH
NW = 32
PER_W = TOTAL_H // NW
IDXROW = 128
CHUNK_ROWS = 4
CHUNK = CHUNK_ROWS * IDXROW
NCHUNK = PER_W // CHUNK


def _gather_body(idx_hbm, table_hbm, out_hbm, idx_v, rows_v, sem):
    wid = lax.axis_index("s") * 2 + lax.axis_index("c")
    row0 = wid * (PER_W // IDXROW)

    def chunk_body(c, carry):
        rb = row0 + c * CHUNK_ROWS
        pltpu.sync_copy(idx_hbm.at[pl.ds(rb, CHUNK_ROWS)], idx_v)

        base = rb * IDXROW

        def fix(i, _):
            j = i // (IDXROW // 16)
            k = (i % (IDXROW // 16)) * 16
            pos = base + i * 16 + lax.iota(jnp.int32, 16)
            off = (pos % FH) * VOCAB
            idx_v[j, pl.ds(k, 16)] = idx_v[j, pl.ds(k, 16)] + off
            return 0

        lax.fori_loop(0, CHUNK_ROWS * (IDXROW // 16), fix, 0)

        descs = [
            pltpu.async_copy(table_hbm.at[idx_v.at[j]],
                             rows_v.at[pl.ds(j * IDXROW, IDXROW)], sem)
            for j in range(CHUNK_ROWS)
        ]
        for d in descs:
            d.wait()
        pltpu.sync_copy(rows_v, out_hbm.at[pl.ds(rb * IDXROW, CHUNK)])
        return carry

    lax.fori_loop(0, NCHUNK, chunk_body, 0)


@functools.lru_cache(maxsize=None)
def _make_gather():
    return pl.kernel(
        _gather_body,
        mesh=plsc.VectorSubcoreMesh(core_axis_name="c", subcore_axis_name="s"),
        out_type=jax.ShapeDtypeStruct((TOTAL_H, EMB), jnp.float32),
        scratch_types=[
            pltpu.VMEM((CHUNK_ROWS, IDXROW), jnp.int32),
            pltpu.VMEM((CHUNK, EMB), jnp.float32),
            pltpu.SemaphoreType.DMA,
        ],
        compiler_params=pltpu.CompilerParams(use_tc_tiling_on_sc=False),
    )


TILE_B = 512
EMB_H = FH * EMB


def _wide_deep_body(dense, onehot, emb1, emb2, wwd, wwo, bw,
                    W1d, W1e1, W1e2, b1, W2, b2, W3, b3, w4t, b4, out):
    f32 = jnp.float32
    d = dense[...]
    wide = (jnp.sum(d * wwd[...], axis=1, keepdims=True)
            + jnp.sum(onehot[...] * wwo[...], axis=1, keepdims=True)
            + bw[...])
    h = jnp.maximum(jnp.dot(d, W1d[...], preferred_element_type=f32)
                    + jnp.dot(emb1[...], W1e1[...], preferred_element_type=f32)
                    + jnp.dot(emb2[...], W1e2[...], preferred_element_type=f32)
                    + b1[...], 0.0)
    h = jnp.maximum(jnp.dot(h, W2[...], preferred_element_type=f32) + b2[...], 0.0)
    h = jnp.maximum(jnp.dot(h, W3[...], preferred_element_type=f32) + b3[...], 0.0)
    deep = jnp.sum(h * w4t[...], axis=1, keepdims=True) + b4[...]
    out[...] = jax.nn.sigmoid(0.5 * (wide + deep))


def _full(shape):
    return pl.BlockSpec(shape, lambda i: (0, 0))


_wide_deep = pl.pallas_call(
    _wide_deep_body,
    grid=(B // TILE_B,),
    in_specs=[
        pl.BlockSpec((TILE_B, F_DENSE), lambda i: (i, 0)),
        pl.BlockSpec((TILE_B, ONEHOT), lambda i: (i, 0)),
        pl.BlockSpec((TILE_B, EMB_H), lambda i: (i, 0)),
        pl.BlockSpec((TILE_B, EMB_H), lambda i: (i, 0)),
        _full((1, F_DENSE)),
        _full((1, ONEHOT)),
        _full((1, 1)),
        _full((F_DENSE, 1024)),
        _full((EMB_H, 1024)),
        _full((EMB_H, 1024)),
        _full((1, 1024)),
        _full((1024, 512)),
        _full((1, 512)),
        _full((512, 256)),
        _full((1, 256)),
        _full((1, 256)),
        _full((1, 1)),
    ],
    out_specs=pl.BlockSpec((TILE_B, 1), lambda i: (i, 0)),
    out_shape=jax.ShapeDtypeStruct((B, 1), jnp.float32),
)


def kernel(dense_inputs, sparse_inputs, onehot_inputs, embed_tables,
           w_wide, b_wide, W1, b1, W2, b2, W3, b3, W4, b4):
    sp = sparse_inputs.astype(jnp.int32)
    gather = _make_gather()
    idx1 = sp[:, :FH].reshape(TOTAL_H // IDXROW, IDXROW)
    idx2 = sp[:, FH:].reshape(TOTAL_H // IDXROW, IDXROW)
    tbl1 = embed_tables[:FH].reshape(FH * VOCAB, EMB)
    tbl2 = embed_tables[FH:].reshape(FH * VOCAB, EMB)
    emb1 = gather(idx1, tbl1).reshape(B, EMB_H)
    emb2 = gather(idx2, tbl2).reshape(B, EMB_H)
    out = _wide_deep(
        dense_inputs, onehot_inputs, emb1, emb2,
        w_wide[:F_DENSE].reshape(1, -1), w_wide[F_DENSE:].reshape(1, -1),
        b_wide.reshape(1, 1),
        W1[:F_DENSE], W1[F_DENSE:F_DENSE + EMB_H], W1[F_DENSE + EMB_H:],
        b1.reshape(1, -1),
        W2, b2.reshape(1, -1), W3, b3.reshape(1, -1),
        W4.reshape(1, -1), b4.reshape(1, 1),
    )
    return out


# speedup vs baseline: 1.4177x; 1.4177x over previous
import functools

import jax
import jax.numpy as jnp
from jax import lax
from jax.experimental import pallas as pl
from jax.experimental.pallas import tpu as pltpu
from jax.experimental.pallas import tpu_sc as plsc

B = 16384
F_DENSE = 13
F_SPARSE = 26
VOCAB = 100000
EMB = 16
ONEHOT = 2600
TOTAL = B * F_SPARSE
NW = 32
PER_W = TOTAL // NW
IDXROW = 128
CHUNK_ROWS = 8
CHUNK = CHUNK_ROWS * IDXROW
NCHUNK = PER_W // CHUNK


def _gather_body(idx_hbm, table_hbm, out_hbm, idx_v, rows_v, out_v, sem):
    wid = lax.axis_index("s") * 2 + lax.axis_index("c")
    row0 = wid * (PER_W // IDXROW)

    def chunk_body(c, carry):
        rb = row0 + c * CHUNK_ROWS
        pltpu.sync_copy(idx_hbm.at[pl.ds(rb, CHUNK_ROWS)], idx_v)

        base = rb * IDXROW

        def fix(i, _):
            j = i // (IDXROW // 16)
            k = (i % (IDXROW // 16)) * 16
            pos = base + i * 16 + lax.iota(jnp.int32, 16)
            off = (pos % F_SPARSE) * VOCAB
            idx_v[j, pl.ds(k, 16)] = idx_v[j, pl.ds(k, 16)] + off
            return 0

        lax.fori_loop(0, CHUNK_ROWS * (IDXROW // 16), fix, 0)

        descs = [
            pltpu.async_copy(table_hbm.at[idx_v.at[j]],
                             rows_v.at[pl.ds(j * IDXROW, IDXROW)], sem)
            for j in range(CHUNK_ROWS)
        ]
        for d in descs:
            d.wait()

        def repack(i, _):
            p = i * EMB
            out_v[p // 128, pl.ds(p % 128, EMB)] = rows_v[i, :]
            return 0

        lax.fori_loop(0, CHUNK, repack, 0)
        pltpu.sync_copy(
            out_v, out_hbm.at[pl.ds(rb * (IDXROW * EMB // 128),
                                    CHUNK * EMB // 128)])
        return carry

    lax.fori_loop(0, NCHUNK, chunk_body, 0)


@functools.lru_cache(maxsize=None)
def _make_gather():
    return pl.kernel(
        _gather_body,
        mesh=plsc.VectorSubcoreMesh(core_axis_name="c", subcore_axis_name="s"),
        out_type=jax.ShapeDtypeStruct((TOTAL * EMB // 128, 128), jnp.float32),
        scratch_types=[
            pltpu.VMEM((CHUNK_ROWS, IDXROW), jnp.int32),
            pltpu.VMEM((CHUNK, EMB), jnp.float32),
            pltpu.VMEM((CHUNK * EMB // 128, 128), jnp.float32),
            pltpu.SemaphoreType.DMA,
        ],
        compiler_params=pltpu.CompilerParams(use_tc_tiling_on_sc=False),
    )


TILE_B = 512


def _wide_deep_body(dense, onehot, embed, wwd, wwo, bw,
                    W1d, W1e, b1, W2, b2, W3, b3, w4t, b4, out):
    f32 = jnp.float32
    d = dense[...]
    wide = (jnp.sum(d * wwd[...], axis=1, keepdims=True)
            + jnp.sum(onehot[...] * wwo[...], axis=1, keepdims=True)
            + bw[...])
    h = jnp.maximum(jnp.dot(d, W1d[...], preferred_element_type=f32)
                    + jnp.dot(embed[...], W1e[...], preferred_element_type=f32)
                    + b1[...], 0.0)
    h = jnp.maximum(jnp.dot(h, W2[...], preferred_element_type=f32) + b2[...], 0.0)
    h = jnp.maximum(jnp.dot(h, W3[...], preferred_element_type=f32) + b3[...], 0.0)
    deep = jnp.sum(h * w4t[...], axis=1, keepdims=True) + b4[...]
    out[...] = jax.nn.sigmoid(0.5 * (wide + deep))


def _full(shape):
    return pl.BlockSpec(shape, lambda i: (0, 0))


_wide_deep = pl.pallas_call(
    _wide_deep_body,
    grid=(B // TILE_B,),
    in_specs=[
        pl.BlockSpec((TILE_B, F_DENSE), lambda i: (i, 0)),
        pl.BlockSpec((TILE_B, ONEHOT), lambda i: (i, 0)),
        pl.BlockSpec((TILE_B, F_SPARSE * EMB), lambda i: (i, 0)),
        _full((1, F_DENSE)),
        _full((1, ONEHOT)),
        _full((1, 1)),
        _full((F_DENSE, 1024)),
        _full((F_SPARSE * EMB, 1024)),
        _full((1, 1024)),
        _full((1024, 512)),
        _full((1, 512)),
        _full((512, 256)),
        _full((1, 256)),
        _full((1, 256)),
        _full((1, 1)),
    ],
    out_specs=pl.BlockSpec((TILE_B, 1), lambda i: (i, 0)),
    out_shape=jax.ShapeDtypeStruct((B, 1), jnp.float32),
)


def kernel(dense_inputs, sparse_inputs, onehot_inputs, embed_tables,
           w_wide, b_wide, W1, b1, W2, b2, W3, b3, W4, b4):
    idx = sparse_inputs.astype(jnp.int32).reshape(TOTAL // IDXROW, IDXROW)
    table = embed_tables.reshape(F_SPARSE * VOCAB, EMB)
    rows = _make_gather()(idx, table)
    embed = rows.reshape(B, F_SPARSE * EMB)
    out = _wide_deep(
        dense_inputs, onehot_inputs, embed,
        w_wide[:F_DENSE].reshape(1, -1), w_wide[F_DENSE:].reshape(1, -1),
        b_wide.reshape(1, 1),
        W1[:F_DENSE], W1[F_DENSE:], b1.reshape(1, -1),
        W2, b2.reshape(1, -1), W3, b3.reshape(1, -1),
        W4.reshape(1, -1), b4.reshape(1, 1),
    )
    return out


# speedup vs baseline: 1.4264x; 1.0061x over previous
import functools

import jax
import jax.numpy as jnp
from jax import lax
from jax.experimental import pallas as pl
from jax.experimental.pallas import tpu as pltpu
from jax.experimental.pallas import tpu_sc as plsc

B = 16384
F_DENSE = 13
F_SPARSE = 26
VOCAB = 100000
EMB = 16
ONEHOT = 2600
TOTAL = B * F_SPARSE
NW = 32
PER_W = TOTAL // NW
IDXROW = 128
CHUNK_ROWS = 8
CHUNK = CHUNK_ROWS * IDXROW
NCHUNK = PER_W // CHUNK


def _gather_body(idx_hbm, table_hbm, out_hbm, idx_v, rows_v, out_v, sem):
    wid = lax.axis_index("s") * 2 + lax.axis_index("c")
    row0 = wid * (PER_W // IDXROW)

    def chunk_body(c, carry):
        rb = row0 + c * CHUNK_ROWS
        pltpu.sync_copy(idx_hbm.at[pl.ds(rb, CHUNK_ROWS)], idx_v)

        base = rb * IDXROW

        def fix(i, _):
            j = i // (IDXROW // 16)
            k = (i % (IDXROW // 16)) * 16
            pos = base + i * 16 + lax.iota(jnp.int32, 16)
            off = (pos % F_SPARSE) * VOCAB
            idx_v[j, pl.ds(k, 16)] = idx_v[j, pl.ds(k, 16)] + off
            return 0

        lax.fori_loop(0, CHUNK_ROWS * (IDXROW // 16), fix, 0)

        descs = [
            pltpu.async_copy(table_hbm.at[idx_v.at[j]],
                             rows_v.at[pl.ds(j * IDXROW, IDXROW)], sem)
            for j in range(CHUNK_ROWS)
        ]
        for d in descs:
            d.wait()

        def repack(i, _):
            p = i * EMB
            out_v[p // 128, pl.ds(p % 128, EMB)] = rows_v[i, :]
            return 0

        lax.fori_loop(0, CHUNK, repack, 0)
        pltpu.sync_copy(
            out_v, out_hbm.at[pl.ds(rb * (IDXROW * EMB // 128),
                                    CHUNK * EMB // 128)])
        return carry

    lax.fori_loop(0, NCHUNK, chunk_body, 0)


@functools.lru_cache(maxsize=None)
def _make_gather():
    return pl.kernel(
        _gather_body,
        mesh=plsc.VectorSubcoreMesh(core_axis_name="c", subcore_axis_name="s"),
        out_type=jax.ShapeDtypeStruct((TOTAL * EMB // 128, 128), jnp.float32),
        scratch_types=[
            pltpu.VMEM((CHUNK_ROWS, IDXROW), jnp.int32),
            pltpu.VMEM((CHUNK, EMB), jnp.float32),
            pltpu.VMEM((CHUNK * EMB // 128, 128), jnp.float32),
            pltpu.SemaphoreType.DMA,
        ],
        compiler_params=pltpu.CompilerParams(use_tc_tiling_on_sc=False),
    )


TILE_B = 1024


def _wide_deep_body(dense, onehot, embed, wwd, wwo, bw,
                    W1d, W1e, b1, W2, b2, W3, b3, w4t, b4, out):
    f32 = jnp.float32
    d = dense[...]
    wide = (jnp.sum(d * wwd[...], axis=1, keepdims=True)
            + jnp.sum(onehot[...] * wwo[...], axis=1, keepdims=True)
            + bw[...])
    h = jnp.maximum(jnp.dot(d, W1d[...], preferred_element_type=f32)
                    + jnp.dot(embed[...], W1e[...], preferred_element_type=f32)
                    + b1[...], 0.0)
    h = jnp.maximum(jnp.dot(h, W2[...], preferred_element_type=f32) + b2[...], 0.0)
    h = jnp.maximum(jnp.dot(h, W3[...], preferred_element_type=f32) + b3[...], 0.0)
    deep = jnp.sum(h * w4t[...], axis=1, keepdims=True) + b4[...]
    out[...] = jax.nn.sigmoid(0.5 * (wide + deep))


def _full(shape):
    return pl.BlockSpec(shape, lambda i: (0, 0))


_wide_deep = pl.pallas_call(
    _wide_deep_body,
    grid=(B // TILE_B,),
    in_specs=[
        pl.BlockSpec((TILE_B, F_DENSE), lambda i: (i, 0)),
        pl.BlockSpec((TILE_B, ONEHOT), lambda i: (i, 0)),
        pl.BlockSpec((TILE_B, F_SPARSE * EMB), lambda i: (i, 0)),
        _full((1, F_DENSE)),
        _full((1, ONEHOT)),
        _full((1, 1)),
        _full((F_DENSE, 1024)),
        _full((F_SPARSE * EMB, 1024)),
        _full((1, 1024)),
        _full((1024, 512)),
        _full((1, 512)),
        _full((512, 256)),
        _full((1, 256)),
        _full((1, 256)),
        _full((1, 1)),
    ],
    out_specs=pl.BlockSpec((TILE_B, 1), lambda i: (i, 0)),
    out_shape=jax.ShapeDtypeStruct((B, 1), jnp.float32),
)


def kernel(dense_inputs, sparse_inputs, onehot_inputs, embed_tables,
           w_wide, b_wide, W1, b1, W2, b2, W3, b3, W4, b4):
    idx = sparse_inputs.astype(jnp.int32).reshape(TOTAL // IDXROW, IDXROW)
    table = embed_tables.reshape(F_SPARSE * VOCAB, EMB)
    rows = _make_gather()(idx, table)
    embed = rows.reshape(B, F_SPARSE * EMB)
    out = _wide_deep(
        dense_inputs, onehot_inputs, embed,
        w_wide[:F_DENSE].reshape(1, -1), w_wide[F_DENSE:].reshape(1, -1),
        b_wide.reshape(1, 1),
        W1[:F_DENSE], W1[F_DENSE:], b1.reshape(1, -1),
        W2, b2.reshape(1, -1), W3, b3.reshape(1, -1),
        W4.reshape(1, -1), b4.reshape(1, 1),
    )
    return out
